# baseline (speedup 1.0000x reference)
import functools

import jax
import jax.numpy as jnp
from jax import lax
from jax.experimental import pallas as pl
from jax.experimental.pallas import tpu as pltpu
from jax.experimental.pallas import tpu_sc as plsc

N = 10000
NPAD = 10240
E = 320000
D = 128
DH = 64
NUM_ET = 40
ET_PAD = 128

L = 16
NC = 2
NS = 16
NW = NC * NS
EPT = E // NW

C = 80
NCHUNK = EPT // C
VPC = C // L
UN = 16

SLC = NPAD // NS

_SELU_LAM = 1.0507009873554804934193349852946
_SELU_ALPHA = 1.6732632423543772848170429916717

_mesh = plsc.VectorSubcoreMesh(core_axis_name="c", subcore_axis_name="s")
_sc_params = pltpu.CompilerParams(
    needs_layout_passes=False, use_tc_tiling_on_sc=False)


def _tc_proj_body(x_ref, w_ref, a2_ref, wr_ref, rel_ref, ae_ref,
                  ha_ref, hb_ref, s_ref, re_ref):
    h = jnp.dot(x_ref[...], w_ref[...], preferred_element_type=jnp.float32)
    ha_ref[...] = h[:, :DH]
    hb_ref[...] = h[:, DH:]
    s_ref[...] = jnp.dot(h, a2_ref[...], preferred_element_type=jnp.float32)
    rp = jnp.dot(rel_ref[...], wr_ref[...], preferred_element_type=jnp.float32)
    re_ref[...] = jnp.dot(rp, ae_ref[...], preferred_element_type=jnp.float32)


_tc_proj = pl.pallas_call(
    _tc_proj_body,
    out_shape=[
        jax.ShapeDtypeStruct((N, DH), jnp.float32),
        jax.ShapeDtypeStruct((N, DH), jnp.float32),
        jax.ShapeDtypeStruct((N, 8), jnp.float32),
        jax.ShapeDtypeStruct((NUM_ET, 8), jnp.float32),
    ],
)


def _zero_vmem_rows(buf, nvecs):
    kv = DH // L

    def zbody(i, carry):
        buf[i // kv, pl.ds((i % kv) * L, L)] = jnp.zeros((L,), jnp.float32)
        return carry

    lax.fori_loop(0, nvecs, zbody, 0)


def _msg_pipeline(src_v, dst_v, et_v, hl_v, hr_v, re_v, ex_v,
                  rows_bufs, sbufs, gsems, ssems, h_hbm, acc_sh,
                  psum_sh=None, sem_p=None):
    def start_gather(c, b):
        pltpu.async_copy(
            h_hbm.at[src_v.at[pl.ds(c * C, C)]], rows_bufs[b], gsems[b])

    def start_scatter(c, b):
        pltpu.async_copy(
            sbufs[b], acc_sh.at[dst_v.at[c]], ssems[b], add=True)

    def wait_gather(b):
        pltpu.make_async_copy(
            h_hbm.at[src_v.at[pl.ds(0, C)]], rows_bufs[b], gsems[b]).wait()

    def wait_scatter(b):
        pltpu.make_async_copy(
            sbufs[b], acc_sh.at[dst_v.at[0]], ssems[b]).wait()

    def compute(c, b):
        rows = rows_bufs[b]
        sb = sbufs[b]

        for j in range(VPC):
            sl = pl.ds(c * C + j * L, L)
            sv = src_v[sl]
            dv = dst_v[c, pl.ds(j * L, L)]
            ev = et_v[sl]
            z = (plsc.load_gather(hl_v, [sv])
                 + plsc.load_gather(hr_v, [dv])
                 + plsc.load_gather(re_v, [ev]))
            e = jnp.where(z >= 0.0, z, 0.2 * z)
            ex_v[sl] = jnp.exp(e)

        if psum_sh is not None:
            pltpu.async_copy(
                ex_v.at[pl.ds(pl.multiple_of(c * C, 8), C)],
                psum_sh.at[dst_v.at[c]], sem_p, add=True)

        @plsc.parallel_loop(0, C, 1, unroll=UN)
        def _(e):
            av = plsc.load_gather(ex_v, [lax.broadcast(c * C + e, (L,))])
            for k in range(DH // L):
                sl = pl.ds(k * L, L)
                sb[e, sl] = rows[e, sl] * av

    start_gather(0, 0)
    start_gather(1, 1)

    def gbody(g, carry):
        for b in range(2):
            c = 2 * g + b
            wait_gather(b)

            @pl.when(g >= 1)
            def _():
                wait_scatter(b)

            compute(c, b)

            @pl.when(c + 2 < NCHUNK)
            def _():
                start_gather(c + 2, b)

            start_scatter(c, b)
        return carry

    lax.fori_loop(0, (NCHUNK - 1) // 2, gbody, 0)

    ct = NCHUNK - 1
    wait_gather(0)
    wait_scatter(0)
    compute(ct, 0)
    start_scatter(ct, 0)
    wait_scatter(1)
    wait_scatter(0)


_msg_scratch = [
    pltpu.VMEM((EPT,), jnp.int32),
    pltpu.VMEM((NCHUNK, C), jnp.int32),
    pltpu.VMEM((EPT,), jnp.int32),
    pltpu.VMEM((NPAD,), jnp.float32),
    pltpu.VMEM((NPAD,), jnp.float32),
    pltpu.VMEM((ET_PAD,), jnp.float32),
    pltpu.VMEM((EPT,), jnp.float32),
    pltpu.VMEM((C, DH), jnp.float32),
    pltpu.VMEM((C, DH), jnp.float32),
    pltpu.VMEM((C, DH), jnp.float32),
    pltpu.VMEM((C, DH), jnp.float32),
    pltpu.VMEM_SHARED((NPAD, DH), jnp.float32),
    pltpu.SemaphoreType.DMA,
    pltpu.SemaphoreType.DMA,
    pltpu.SemaphoreType.DMA,
    pltpu.SemaphoreType.DMA,
]


@functools.partial(
    pl.kernel,
    out_type=[
        jax.ShapeDtypeStruct((NC, NPAD, DH), jnp.float32),
        jax.ShapeDtypeStruct((NC, NPAD), jnp.float32),
    ],
    mesh=_mesh,
    compiler_params=_sc_params,
    scratch_types=_msg_scratch + [
        pltpu.VMEM_SHARED((NPAD,), jnp.float32),
        pltpu.SemaphoreType.DMA,
    ],
)
def _sc_msg_a(src_hbm, dstr_hbm, et_hbm, hl_hbm, hr_hbm, re_hbm, h_hbm,
              agg_hbm, psum_hbm,
              src_v, dst_v, et_v, hl_v, hr_v, re_v, ex_v,
              rows0_v, rows1_v, sb0_v, sb1_v, acc_sh,
              sem_g0, sem_g1, sem_s0, sem_s1, psum_sh, sem_p):
    cid = lax.axis_index("c")
    sid = lax.axis_index("s")
    w = sid * NC + cid
    base = pl.multiple_of(w * EPT, 8)

    pltpu.sync_copy(src_hbm.at[pl.ds(base, EPT)], src_v)
    pltpu.sync_copy(dstr_hbm.at[w], dst_v)
    pltpu.sync_copy(et_hbm.at[pl.ds(base, EPT)], et_v)
    pltpu.sync_copy(hl_hbm, hl_v)
    pltpu.sync_copy(hr_hbm, hr_v)
    pltpu.sync_copy(re_hbm, re_v)

    _zero_vmem_rows(sb0_v, C * DH // L)
    rbase = pl.multiple_of(sid * SLC, 8)
    for t in range(SLC // C):
        pltpu.sync_copy(sb0_v, acc_sh.at[pl.ds(rbase + t * C, C), :])

    def zex(i, carry):
        ex_v[pl.ds(i * L, L)] = jnp.zeros((L,), jnp.float32)
        return carry

    lax.fori_loop(0, SLC // L, zex, 0)
    pltpu.sync_copy(ex_v.at[pl.ds(0, SLC)], psum_sh.at[pl.ds(rbase, SLC)])
    plsc.subcore_barrier()

    _msg_pipeline(src_v, dst_v, et_v, hl_v, hr_v, re_v, ex_v,
                  (rows0_v, rows1_v), (sb0_v, sb1_v),
                  (sem_g0, sem_g1), (sem_s0, sem_s1), h_hbm, acc_sh,
                  psum_sh=psum_sh, sem_p=sem_p)

    def drain_p(c, carry):
        pltpu.make_async_copy(
            ex_v.at[pl.ds(0, C)], psum_sh.at[dst_v.at[0]], sem_p).wait()
        return carry

    lax.fori_loop(0, NCHUNK, drain_p, 0)
    plsc.subcore_barrier()

    pltpu.sync_copy(acc_sh.at[pl.ds(rbase, SLC), :],
                    agg_hbm.at[cid, pl.ds(rbase, SLC), :])
    pltpu.sync_copy(psum_sh.at[pl.ds(rbase, SLC)],
                    psum_hbm.at[cid, pl.ds(rbase, SLC)])


@functools.partial(
    pl.kernel,
    out_type=jax.ShapeDtypeStruct((NC, NPAD, DH), jnp.float32),
    mesh=_mesh,
    compiler_params=_sc_params,
    scratch_types=_msg_scratch,
)
def _sc_msg_b(src_hbm, dstr_hbm, et_hbm, hl_hbm, hr_hbm, re_hbm, h_hbm,
              agg_hbm,
              src_v, dst_v, et_v, hl_v, hr_v, re_v, ex_v,
              rows0_v, rows1_v, sb0_v, sb1_v, acc_sh,
              sem_g0, sem_g1, sem_s0, sem_s1):
    cid = lax.axis_index("c")
    sid = lax.axis_index("s")
    w = sid * NC + cid
    base = pl.multiple_of(w * EPT, 8)

    pltpu.sync_copy(src_hbm.at[pl.ds(base, EPT)], src_v)
    pltpu.sync_copy(dstr_hbm.at[w], dst_v)
    pltpu.sync_copy(et_hbm.at[pl.ds(base, EPT)], et_v)
    pltpu.sync_copy(hl_hbm, hl_v)
    pltpu.sync_copy(hr_hbm, hr_v)
    pltpu.sync_copy(re_hbm, re_v)

    _zero_vmem_rows(sb0_v, C * DH // L)
    rbase = pl.multiple_of(sid * SLC, 8)
    for t in range(SLC // C):
        pltpu.sync_copy(sb0_v, acc_sh.at[pl.ds(rbase + t * C, C), :])
    plsc.subcore_barrier()

    _msg_pipeline(src_v, dst_v, et_v, hl_v, hr_v, re_v, ex_v,
                  (rows0_v, rows1_v), (sb0_v, sb1_v),
                  (sem_g0, sem_g1), (sem_s0, sem_s1), h_hbm, acc_sh)

    plsc.subcore_barrier()
    pltpu.sync_copy(acc_sh.at[pl.ds(rbase, SLC), :],
                    agg_hbm.at[cid, pl.ds(rbase, SLC), :])


def _tc_fin_body(a_ref, b_ref, p_ref, o_ref):
    s = p_ref[0] + p_ref[1] + 1e-16
    x = jnp.concatenate([a_ref[0] + a_ref[1], b_ref[0] + b_ref[1]], axis=-1)
    x = x / s[:, None]
    o_ref[...] = _SELU_LAM * jnp.where(
        x > 0.0, x, _SELU_ALPHA * (jnp.exp(x) - 1.0))


_tc_fin = pl.pallas_call(
    _tc_fin_body,
    out_shape=jax.ShapeDtypeStruct((NPAD, D), jnp.float32),
)


def kernel(node_features, edge_index, edge_type, W, W_r, rel_emb,
           a_l, a_r, a_e):
    src = edge_index[0].astype(jnp.int32)
    dst = edge_index[1].astype(jnp.int32)
    et = edge_type.astype(jnp.int32)
    a2 = jnp.pad(jnp.stack([a_l, a_r], axis=1), ((0, 0), (0, 6)))
    ae8 = jnp.pad(a_e[:, None], ((0, 0), (0, 7)))

    ha, hb, s8, re8 = _tc_proj(node_features, W, a2, W_r, rel_emb, ae8)
    hl = jnp.pad(s8[:, 0], (0, NPAD - N))
    hr = jnp.pad(s8[:, 1], (0, NPAD - N))
    re64 = jnp.pad(re8[:, 0], (0, ET_PAD - NUM_ET))

    dst_r = dst.reshape(NW, NCHUNK, C)
    agg_a, psum = _sc_msg_a(src, dst_r, et, hl, hr, re64, ha)
    agg_b = _sc_msg_b(src, dst_r, et, hl, hr, re64, hb)
    out = _tc_fin(agg_a, agg_b, psum)
    return out[:N]


# speedup vs baseline: 1.0170x; 1.0170x over previous
import functools

import jax
import jax.numpy as jnp
from jax import lax
from jax.experimental import pallas as pl
from jax.experimental.pallas import tpu as pltpu
from jax.experimental.pallas import tpu_sc as plsc

N = 10000
NPAD = 10240
E = 320000
D = 128
DH = 64
NUM_ET = 40
ET_PAD = 128

L = 16
NC = 2
NS = 16
NW = NC * NS
EPT = E // NW

C = 80
NCHUNK = EPT // C
VPC = C // L
UN = 4

SLC = NPAD // NS

_SELU_LAM = 1.0507009873554804934193349852946
_SELU_ALPHA = 1.6732632423543772848170429916717

_mesh = plsc.VectorSubcoreMesh(core_axis_name="c", subcore_axis_name="s")
_sc_params = pltpu.CompilerParams(
    needs_layout_passes=False, use_tc_tiling_on_sc=False)


def _tc_proj_body(x_ref, w_ref, a2_ref, wr_ref, rel_ref, ae_ref,
                  ha_ref, hb_ref, s_ref, re_ref):
    h = jnp.dot(x_ref[...], w_ref[...], preferred_element_type=jnp.float32)
    ha_ref[...] = h[:, :DH]
    hb_ref[...] = h[:, DH:]
    s_ref[...] = jnp.dot(h, a2_ref[...], preferred_element_type=jnp.float32)
    rp = jnp.dot(rel_ref[...], wr_ref[...], preferred_element_type=jnp.float32)
    re_ref[...] = jnp.dot(rp, ae_ref[...], preferred_element_type=jnp.float32)


_tc_proj = pl.pallas_call(
    _tc_proj_body,
    out_shape=[
        jax.ShapeDtypeStruct((N, DH), jnp.float32),
        jax.ShapeDtypeStruct((N, DH), jnp.float32),
        jax.ShapeDtypeStruct((N, 8), jnp.float32),
        jax.ShapeDtypeStruct((NUM_ET, 8), jnp.float32),
    ],
)


def _zero_vmem_rows(buf, nvecs):
    kv = DH // L

    def zbody(i, carry):
        buf[i // kv, pl.ds((i % kv) * L, L)] = jnp.zeros((L,), jnp.float32)
        return carry

    lax.fori_loop(0, nvecs, zbody, 0)


def _msg_pipeline(src_v, dst_v, et_v, hl_v, hr_v, re_v, ex_v,
                  rows_bufs, sbufs, gsems, ssems, h_hbm, acc_sh,
                  psum_sh=None, sem_p=None):
    def start_gather(c, b):
        pltpu.async_copy(
            h_hbm.at[src_v.at[pl.ds(c * C, C)]], rows_bufs[b], gsems[b])

    def start_scatter(c, b):
        pltpu.async_copy(
            sbufs[b], acc_sh.at[dst_v.at[c]], ssems[b], add=True)

    def wait_gather(b):
        pltpu.make_async_copy(
            h_hbm.at[src_v.at[pl.ds(0, C)]], rows_bufs[b], gsems[b]).wait()

    def wait_scatter(b):
        pltpu.make_async_copy(
            sbufs[b], acc_sh.at[dst_v.at[0]], ssems[b]).wait()

    def compute(c, b):
        rows = rows_bufs[b]
        sb = sbufs[b]

        for j in range(VPC):
            sl = pl.ds(c * C + j * L, L)
            sv = src_v[sl]
            dv = dst_v[c, pl.ds(j * L, L)]
            ev = et_v[sl]
            z = (plsc.load_gather(hl_v, [sv])
                 + plsc.load_gather(hr_v, [dv])
                 + plsc.load_gather(re_v, [ev]))
            e = jnp.where(z >= 0.0, z, 0.2 * z)
            ex_v[sl] = jnp.exp(e)

        if psum_sh is not None:
            pltpu.async_copy(
                ex_v.at[pl.ds(pl.multiple_of(c * C, 8), C)],
                psum_sh.at[dst_v.at[c]], sem_p, add=True)

        @plsc.parallel_loop(0, C, 1, unroll=UN)
        def _(e):
            av = plsc.load_gather(ex_v, [lax.broadcast(c * C + e, (L,))])
            for k in range(DH // L):
                sl = pl.ds(k * L, L)
                sb[e, sl] = rows[e, sl] * av

    start_gather(0, 0)
    start_gather(1, 1)

    def gbody(g, carry):
        for b in range(2):
            c = 2 * g + b
            wait_gather(b)

            @pl.when(g >= 1)
            def _():
                wait_scatter(b)

            compute(c, b)

            @pl.when(c + 2 < NCHUNK)
            def _():
                start_gather(c + 2, b)

            start_scatter(c, b)
        return carry

    lax.fori_loop(0, (NCHUNK - 1) // 2, gbody, 0)

    ct = NCHUNK - 1
    wait_gather(0)
    wait_scatter(0)
    compute(ct, 0)
    start_scatter(ct, 0)
    wait_scatter(1)
    wait_scatter(0)


_msg_scratch = [
    pltpu.VMEM((EPT,), jnp.int32),
    pltpu.VMEM((NCHUNK, C), jnp.int32),
    pltpu.VMEM((EPT,), jnp.int32),
    pltpu.VMEM((NPAD,), jnp.float32),
    pltpu.VMEM((NPAD,), jnp.float32),
    pltpu.VMEM((ET_PAD,), jnp.float32),
    pltpu.VMEM((EPT,), jnp.float32),
    pltpu.VMEM((C, DH), jnp.float32),
    pltpu.VMEM((C, DH), jnp.float32),
    pltpu.VMEM((C, DH), jnp.float32),
    pltpu.VMEM((C, DH), jnp.float32),
    pltpu.VMEM_SHARED((NPAD, DH), jnp.float32),
    pltpu.SemaphoreType.DMA,
    pltpu.SemaphoreType.DMA,
    pltpu.SemaphoreType.DMA,
    pltpu.SemaphoreType.DMA,
]


@functools.partial(
    pl.kernel,
    out_type=[
        jax.ShapeDtypeStruct((NC, NPAD, DH), jnp.float32),
        jax.ShapeDtypeStruct((NC, NPAD), jnp.float32),
    ],
    mesh=_mesh,
    compiler_params=_sc_params,
    scratch_types=_msg_scratch + [
        pltpu.VMEM_SHARED((NPAD,), jnp.float32),
        pltpu.SemaphoreType.DMA,
    ],
)
def _sc_msg_a(src_hbm, dstr_hbm, et_hbm, hl_hbm, hr_hbm, re_hbm, h_hbm,
              agg_hbm, psum_hbm,
              src_v, dst_v, et_v, hl_v, hr_v, re_v, ex_v,
              rows0_v, rows1_v, sb0_v, sb1_v, acc_sh,
              sem_g0, sem_g1, sem_s0, sem_s1, psum_sh, sem_p):
    cid = lax.axis_index("c")
    sid = lax.axis_index("s")
    w = sid * NC + cid
    base = pl.multiple_of(w * EPT, 8)

    pltpu.sync_copy(src_hbm.at[pl.ds(base, EPT)], src_v)
    pltpu.sync_copy(dstr_hbm.at[w], dst_v)
    pltpu.sync_copy(et_hbm.at[pl.ds(base, EPT)], et_v)
    pltpu.sync_copy(hl_hbm, hl_v)
    pltpu.sync_copy(hr_hbm, hr_v)
    pltpu.sync_copy(re_hbm, re_v)

    _zero_vmem_rows(sb0_v, C * DH // L)
    rbase = pl.multiple_of(sid * SLC, 8)
    for t in range(SLC // C):
        pltpu.sync_copy(sb0_v, acc_sh.at[pl.ds(rbase + t * C, C), :])

    def zex(i, carry):
        ex_v[pl.ds(i * L, L)] = jnp.zeros((L,), jnp.float32)
        return carry

    lax.fori_loop(0, SLC // L, zex, 0)
    pltpu.sync_copy(ex_v.at[pl.ds(0, SLC)], psum_sh.at[pl.ds(rbase, SLC)])
    plsc.subcore_barrier()

    _msg_pipeline(src_v, dst_v, et_v, hl_v, hr_v, re_v, ex_v,
                  (rows0_v, rows1_v), (sb0_v, sb1_v),
                  (sem_g0, sem_g1), (sem_s0, sem_s1), h_hbm, acc_sh,
                  psum_sh=psum_sh, sem_p=sem_p)

    def drain_p(c, carry):
        pltpu.make_async_copy(
            ex_v.at[pl.ds(0, C)], psum_sh.at[dst_v.at[0]], sem_p).wait()
        return carry

    lax.fori_loop(0, NCHUNK, drain_p, 0)
    plsc.subcore_barrier()

    pltpu.sync_copy(acc_sh.at[pl.ds(rbase, SLC), :],
                    agg_hbm.at[cid, pl.ds(rbase, SLC), :])
    pltpu.sync_copy(psum_sh.at[pl.ds(rbase, SLC)],
                    psum_hbm.at[cid, pl.ds(rbase, SLC)])


@functools.partial(
    pl.kernel,
    out_type=jax.ShapeDtypeStruct((NC, NPAD, DH), jnp.float32),
    mesh=_mesh,
    compiler_params=_sc_params,
    scratch_types=_msg_scratch,
)
def _sc_msg_b(src_hbm, dstr_hbm, et_hbm, hl_hbm, hr_hbm, re_hbm, h_hbm,
              agg_hbm,
              src_v, dst_v, et_v, hl_v, hr_v, re_v, ex_v,
              rows0_v, rows1_v, sb0_v, sb1_v, acc_sh,
              sem_g0, sem_g1, sem_s0, sem_s1):
    cid = lax.axis_index("c")
    sid = lax.axis_index("s")
    w = sid * NC + cid
    base = pl.multiple_of(w * EPT, 8)

    pltpu.sync_copy(src_hbm.at[pl.ds(base, EPT)], src_v)
    pltpu.sync_copy(dstr_hbm.at[w], dst_v)
    pltpu.sync_copy(et_hbm.at[pl.ds(base, EPT)], et_v)
    pltpu.sync_copy(hl_hbm, hl_v)
    pltpu.sync_copy(hr_hbm, hr_v)
    pltpu.sync_copy(re_hbm, re_v)

    _zero_vmem_rows(sb0_v, C * DH // L)
    rbase = pl.multiple_of(sid * SLC, 8)
    for t in range(SLC // C):
        pltpu.sync_copy(sb0_v, acc_sh.at[pl.ds(rbase + t * C, C), :])
    plsc.subcore_barrier()

    _msg_pipeline(src_v, dst_v, et_v, hl_v, hr_v, re_v, ex_v,
                  (rows0_v, rows1_v), (sb0_v, sb1_v),
                  (sem_g0, sem_g1), (sem_s0, sem_s1), h_hbm, acc_sh)

    plsc.subcore_barrier()
    pltpu.sync_copy(acc_sh.at[pl.ds(rbase, SLC), :],
                    agg_hbm.at[cid, pl.ds(rbase, SLC), :])


def _tc_fin_body(a_ref, b_ref, p_ref, o_ref):
    s = p_ref[0] + p_ref[1] + 1e-16
    x = jnp.concatenate([a_ref[0] + a_ref[1], b_ref[0] + b_ref[1]], axis=-1)
    x = x / s[:, None]
    o_ref[...] = _SELU_LAM * jnp.where(
        x > 0.0, x, _SELU_ALPHA * (jnp.exp(x) - 1.0))


_tc_fin = pl.pallas_call(
    _tc_fin_body,
    out_shape=jax.ShapeDtypeStruct((NPAD, D), jnp.float32),
)


def kernel(node_features, edge_index, edge_type, W, W_r, rel_emb,
           a_l, a_r, a_e):
    src = edge_index[0].astype(jnp.int32)
    dst = edge_index[1].astype(jnp.int32)
    et = edge_type.astype(jnp.int32)
    a2 = jnp.pad(jnp.stack([a_l, a_r], axis=1), ((0, 0), (0, 6)))
    ae8 = jnp.pad(a_e[:, None], ((0, 0), (0, 7)))

    ha, hb, s8, re8 = _tc_proj(node_features, W, a2, W_r, rel_emb, ae8)
    hl = jnp.pad(s8[:, 0], (0, NPAD - N))
    hr = jnp.pad(s8[:, 1], (0, NPAD - N))
    re64 = jnp.pad(re8[:, 0], (0, ET_PAD - NUM_ET))

    dst_r = dst.reshape(NW, NCHUNK, C)
    agg_a, psum = _sc_msg_a(src, dst_r, et, hl, hr, re64, ha)
    agg_b = _sc_msg_b(src, dst_r, et, hl, hr, re64, hb)
    out = _tc_fin(agg_a, agg_b, psum)
    return out[:N]
